# baseline (speedup 1.0000x reference)
import functools

import jax
import jax.numpy as jnp
from jax import lax
from jax.experimental import pallas as pl
from jax.experimental.pallas import tpu as pltpu
from jax.experimental.pallas import tpu_sc as plsc

VOCAB = 1000000
SEQ = 200
EMBED = 64
BATCH = 4096

_NC = 2
_NS = 16
_NW = _NC * _NS

_BBLK = BATCH // _NW
_CH_S = 2
_CHUNK = _CH_S * _BBLK
_CHUNKS = SEQ // _CH_S
_LANES = 16
_EG = EMBED // _LANES
_ETI = EMBED // 8


@functools.partial(
    pl.kernel,
    mesh=plsc.VectorSubcoreMesh(core_axis_name="c", subcore_axis_name="s"),
    compiler_params=pltpu.CompilerParams(use_tc_tiling_on_sc=False,
                                         needs_layout_passes=False),
    out_type=jax.ShapeDtypeStruct((SEQ, _ETI, _NW, 8, _BBLK), jnp.float32),
    scratch_types=[
        pltpu.VMEM((_CH_S, _BBLK), jnp.int32),
        pltpu.VMEM((_CH_S, _BBLK), jnp.int32),
        pltpu.VMEM((_CHUNK, EMBED), jnp.float32),
        pltpu.VMEM((_CHUNK, EMBED), jnp.float32),
        pltpu.VMEM((_CH_S, _ETI, 1, 8, _BBLK), jnp.float32),
        pltpu.VMEM((_CH_S, _ETI, 1, 8, _BBLK), jnp.float32),
        pltpu.VMEM((SEQ, EMBED), jnp.float32),
        pltpu.SemaphoreType.DMA,
        pltpu.SemaphoreType.DMA,
        pltpu.SemaphoreType.DMA,
        pltpu.SemaphoreType.DMA,
        pltpu.SemaphoreType.DMA,
        pltpu.SemaphoreType.DMA,
    ],
)
def _emb_kernel(idx_hbm, tok_hbm, pos_hbm, out_hbm,
                idx0, idx1, rows0, rows1, tr0, tr1, pos_v,
                isem0, isem1, gsem0, gsem1, ssem0, ssem1):
    wid = lax.axis_index("s") * _NC + lax.axis_index("c")
    bbase = wid * _BBLK
    pltpu.sync_copy(pos_hbm, pos_v)

    idx = (idx0, idx1)
    rows = (rows0, rows1)
    trans = (tr0, tr1)
    isem = (isem0, isem1)
    gsem = (gsem0, gsem1)
    ssem = (ssem0, ssem1)

    def fire_idx(c, buf):
        for s_loc in range(_CH_S):
            pltpu.async_copy(
                idx_hbm.at[pl.ds((c * _CH_S + s_loc) * BATCH + bbase, _BBLK)],
                idx[buf].at[s_loc], isem[buf])

    def wait_idx(buf):
        for s_loc in range(_CH_S):
            pltpu.make_async_copy(idx_hbm.at[pl.ds(0, _BBLK)],
                                  idx[buf].at[s_loc], isem[buf]).wait()

    def fire_gathers(buf):
        for s_loc in range(_CH_S):
            pltpu.async_copy(tok_hbm.at[idx[buf].at[s_loc]],
                             rows[buf].at[pl.ds(s_loc * _BBLK, _BBLK)],
                             gsem[buf])

    def wait_gathers(buf):
        for s_loc in range(_CH_S):
            pltpu.make_async_copy(tok_hbm.at[idx[buf].at[s_loc]],
                                  rows[buf].at[pl.ds(s_loc * _BBLK, _BBLK)],
                                  gsem[buf]).wait()

    def fire_scatter(c, buf):
        pltpu.async_copy(trans[buf],
                         out_hbm.at[pl.ds(c * _CH_S, _CH_S), :,
                                    pl.ds(wid, 1)],
                         ssem[buf])

    def wait_scatter(buf):
        pltpu.make_async_copy(trans[buf],
                              out_hbm.at[pl.ds(0, _CH_S), :, pl.ds(0, 1)],
                              ssem[buf]).wait()

    zeros16 = jnp.zeros((_LANES,), jnp.int32)
    lane = lax.iota(jnp.int32, _LANES)
    i_s = [zeros16 + s_loc for s_loc in range(_CH_S)]
    i_et = []
    i_e8 = []
    for eg in range(_EG):
        e_vec = lane + eg * _LANES
        i_et.append(lax.shift_right_logical(e_vec, 3))
        i_e8.append(lax.bitwise_and(e_vec, 7))

    def compute(c, buf):
        r = rows[buf]
        t = trans[buf]
        for s_loc in range(_CH_S):
            s = c * _CH_S + s_loc
            for eg in range(_EG):
                sl = pl.ds(eg * _LANES, _LANES)

                def grp(g, carry, s=s, s_loc=s_loc, eg=eg, sl=sl):
                    pv = pos_v[s, sl]
                    for k in range(4):
                        b = g * 4 + k
                        v = r[s_loc * _BBLK + b, sl] + pv
                        plsc.store_scatter(
                            t, [i_s[s_loc], i_et[eg], zeros16, i_e8[eg],
                                zeros16 + b], v)
                    return carry

                lax.fori_loop(0, _BBLK // 4, grp, 0)

    def step(c, buf, fire_next_idx=True, wait_sc=True):
        obuf = 1 - buf
        wait_gathers(obuf)
        wait_idx(buf)
        fire_gathers(buf)
        if fire_next_idx:
            fire_idx(c + 1, obuf)
        if wait_sc:
            wait_scatter(obuf)
        compute(c - 1, obuf)
        fire_scatter(c - 1, obuf)

    fire_idx(0, 0)
    wait_idx(0)
    fire_idx(1, 1)
    fire_gathers(0)
    step(1, 1, wait_sc=False)
    step(2, 0, wait_sc=False)

    def super_body(i, carry):
        step(2 * i + 1, 1)
        step(2 * i + 2, 0)
        return carry

    lax.fori_loop(1, _CHUNKS // 2 - 1, super_body, 0)

    step(_CHUNKS - 1, 1, fire_next_idx=False)
    wait_gathers(1)
    wait_scatter(1)
    compute(_CHUNKS - 1, 1)
    fire_scatter(_CHUNKS - 1, 1)
    wait_scatter(0)
    wait_scatter(1)


def kernel(inputs, token_table, pos_table):
    idx = jnp.minimum(jnp.maximum(inputs, 0), VOCAB - 1).T.reshape(-1)
    out5 = _emb_kernel(idx, token_table, pos_table)
    return out5.transpose(2, 4, 0, 1, 3).reshape(BATCH, SEQ, EMBED)


# speedup vs baseline: 1.2998x; 1.2998x over previous
import functools

import jax
import jax.numpy as jnp
from jax import lax
from jax.experimental import pallas as pl
from jax.experimental.pallas import tpu as pltpu
from jax.experimental.pallas import tpu_sc as plsc

VOCAB = 1000000
SEQ = 200
EMBED = 64
BATCH = 4096

_NC = 2
_NS = 16
_NW = _NC * _NS

_BBLK = BATCH // _NW
_CH_S = 2
_CHUNK = _CH_S * _BBLK
_CHUNKS = SEQ // _CH_S
_LANES = 16
_EG = EMBED // _LANES
_ETI = EMBED // 8


@functools.partial(
    pl.kernel,
    mesh=plsc.VectorSubcoreMesh(core_axis_name="c", subcore_axis_name="s"),
    compiler_params=pltpu.CompilerParams(use_tc_tiling_on_sc=False,
                                         needs_layout_passes=False),
    out_type=jax.ShapeDtypeStruct((SEQ, _ETI, _NW, 8, _BBLK), jnp.float32),
    scratch_types=[
        pltpu.VMEM((_CH_S, _BBLK), jnp.int32),
        pltpu.VMEM((_CH_S, _BBLK), jnp.int32),
        pltpu.VMEM((_CHUNK, EMBED), jnp.float32),
        pltpu.VMEM((_CHUNK, EMBED), jnp.float32),
        pltpu.VMEM((_CH_S, _ETI, 1, 8, _BBLK), jnp.float32),
        pltpu.VMEM((_CH_S, _ETI, 1, 8, _BBLK), jnp.float32),
        pltpu.VMEM((SEQ, EMBED), jnp.float32),
        pltpu.SemaphoreType.DMA,
        pltpu.SemaphoreType.DMA,
        pltpu.SemaphoreType.DMA,
        pltpu.SemaphoreType.DMA,
        pltpu.SemaphoreType.DMA,
        pltpu.SemaphoreType.DMA,
    ],
)
def _emb_kernel(idx_hbm, tok_hbm, pos_hbm, out_hbm,
                idx0, idx1, rows0, rows1, tr0, tr1, pos_v,
                isem0, isem1, gsem0, gsem1, ssem0, ssem1):
    wid = lax.axis_index("s") * _NC + lax.axis_index("c")
    bbase = wid * _BBLK
    pltpu.sync_copy(pos_hbm, pos_v)

    idx = (idx0, idx1)
    rows = (rows0, rows1)
    trans = (tr0, tr1)
    isem = (isem0, isem1)
    gsem = (gsem0, gsem1)
    ssem = (ssem0, ssem1)

    def fire_idx(c, buf):
        for s_loc in range(_CH_S):
            pltpu.async_copy(
                idx_hbm.at[pl.ds((c * _CH_S + s_loc) * BATCH + bbase, _BBLK)],
                idx[buf].at[s_loc], isem[buf])

    def wait_idx(buf):
        for s_loc in range(_CH_S):
            pltpu.make_async_copy(idx_hbm.at[pl.ds(0, _BBLK)],
                                  idx[buf].at[s_loc], isem[buf]).wait()

    def fire_gathers(buf):
        for s_loc in range(_CH_S):
            pltpu.async_copy(tok_hbm.at[idx[buf].at[s_loc]],
                             rows[buf].at[pl.ds(s_loc * _BBLK, _BBLK)],
                             gsem[buf])

    def wait_gathers(buf):
        for s_loc in range(_CH_S):
            pltpu.make_async_copy(tok_hbm.at[idx[buf].at[s_loc]],
                                  rows[buf].at[pl.ds(s_loc * _BBLK, _BBLK)],
                                  gsem[buf]).wait()

    def fire_scatter(c, buf):
        pltpu.async_copy(trans[buf],
                         out_hbm.at[pl.ds(c * _CH_S, _CH_S), :,
                                    pl.ds(wid, 1)],
                         ssem[buf])

    def wait_scatter(buf):
        pltpu.make_async_copy(trans[buf],
                              out_hbm.at[pl.ds(0, _CH_S), :, pl.ds(0, 1)],
                              ssem[buf]).wait()

    zeros16 = jnp.zeros((_LANES,), jnp.int32)
    lane = lax.iota(jnp.int32, _LANES)
    i_s = [zeros16 + s_loc for s_loc in range(_CH_S)]
    i_et = []
    i_e8 = []
    for eg in range(_EG):
        e_vec = lane + eg * _LANES
        i_et.append(lax.shift_right_logical(e_vec, 3))
        i_e8.append(lax.bitwise_and(e_vec, 7))

    def compute(c, buf):
        r = rows[buf]
        t = trans[buf]
        for s_loc in range(_CH_S):
            s = c * _CH_S + s_loc
            for eg in range(_EG):
                sl = pl.ds(eg * _LANES, _LANES)

                @plsc.parallel_loop(0, _BBLK, 1, unroll=8)
                def body(b, s=s, s_loc=s_loc, eg=eg, sl=sl):
                    pv = pos_v[s, sl]
                    v = r[s_loc * _BBLK + b, sl] + pv
                    plsc.store_scatter(
                        t, [i_s[s_loc], i_et[eg], zeros16, i_e8[eg],
                            zeros16 + b], v)

    def step(c, buf, fire_next_idx=True, wait_sc=True):
        obuf = 1 - buf
        wait_gathers(obuf)
        wait_idx(buf)
        fire_gathers(buf)
        if fire_next_idx:
            fire_idx(c + 1, obuf)
        if wait_sc:
            wait_scatter(obuf)
        compute(c - 1, obuf)
        fire_scatter(c - 1, obuf)

    fire_idx(0, 0)
    wait_idx(0)
    fire_idx(1, 1)
    fire_gathers(0)
    step(1, 1, wait_sc=False)
    step(2, 0, wait_sc=False)

    def super_body(i, carry):
        step(2 * i + 1, 1)
        step(2 * i + 2, 0)
        return carry

    lax.fori_loop(1, _CHUNKS // 2 - 1, super_body, 0)

    step(_CHUNKS - 1, 1, fire_next_idx=False)
    wait_gathers(1)
    wait_scatter(1)
    compute(_CHUNKS - 1, 1)
    fire_scatter(_CHUNKS - 1, 1)
    wait_scatter(0)
    wait_scatter(1)


def kernel(inputs, token_table, pos_table):
    idx = jnp.minimum(jnp.maximum(inputs, 0), VOCAB - 1).T.reshape(-1)
    out5 = _emb_kernel(idx, token_table, pos_table)
    return out5.transpose(2, 4, 0, 1, 3).reshape(BATCH, SEQ, EMBED)


# speedup vs baseline: 2.2435x; 1.7261x over previous
import functools

import jax
import jax.numpy as jnp
from jax import lax
from jax.experimental import pallas as pl
from jax.experimental.pallas import tpu as pltpu
from jax.experimental.pallas import tpu_sc as plsc

VOCAB = 1000000
SEQ = 200
EMBED = 64
BATCH = 4096

_NC = 2
_NS = 16
_NW = _NC * _NS

_BBLK = BATCH // _NW
_CH_S = 2
_CHUNK = _CH_S * _BBLK
_CHUNKS = SEQ // _CH_S
_LANES = 16
_EG = EMBED // _LANES
_ETI = EMBED // 8


@functools.partial(
    pl.kernel,
    mesh=plsc.VectorSubcoreMesh(core_axis_name="c", subcore_axis_name="s"),
    compiler_params=pltpu.CompilerParams(use_tc_tiling_on_sc=False,
                                         needs_layout_passes=False),
    out_type=jax.ShapeDtypeStruct((SEQ, _ETI, _NW, 8, _BBLK), jnp.float32),
    scratch_types=[
        pltpu.VMEM((_CH_S, _BBLK), jnp.int32),
        pltpu.VMEM((_CH_S, _BBLK), jnp.int32),
        pltpu.VMEM((_CHUNK, EMBED), jnp.float32),
        pltpu.VMEM((_CHUNK, EMBED), jnp.float32),
        pltpu.VMEM((_CH_S, _ETI, 1, 8, _BBLK + 1), jnp.float32),
        pltpu.VMEM((_CH_S, _ETI, 1, 8, _BBLK + 1), jnp.float32),
        pltpu.VMEM((SEQ, EMBED), jnp.float32),
        pltpu.SemaphoreType.DMA,
        pltpu.SemaphoreType.DMA,
        pltpu.SemaphoreType.DMA,
        pltpu.SemaphoreType.DMA,
        pltpu.SemaphoreType.DMA,
        pltpu.SemaphoreType.DMA,
    ],
)
def _emb_kernel(idx_hbm, tok_hbm, pos_hbm, out_hbm,
                idx0, idx1, rows0, rows1, tr0, tr1, pos_v,
                isem0, isem1, gsem0, gsem1, ssem0, ssem1):
    wid = lax.axis_index("s") * _NC + lax.axis_index("c")
    bbase = wid * _BBLK
    pltpu.sync_copy(pos_hbm, pos_v)

    idx = (idx0, idx1)
    rows = (rows0, rows1)
    trans = (tr0, tr1)
    isem = (isem0, isem1)
    gsem = (gsem0, gsem1)
    ssem = (ssem0, ssem1)

    def fire_idx(c, buf):
        for s_loc in range(_CH_S):
            pltpu.async_copy(
                idx_hbm.at[pl.ds((c * _CH_S + s_loc) * BATCH + bbase, _BBLK)],
                idx[buf].at[s_loc], isem[buf])

    def wait_idx(buf):
        for s_loc in range(_CH_S):
            pltpu.make_async_copy(idx_hbm.at[pl.ds(0, _BBLK)],
                                  idx[buf].at[s_loc], isem[buf]).wait()

    def fire_gathers(buf):
        for s_loc in range(_CH_S):
            pltpu.async_copy(tok_hbm.at[idx[buf].at[s_loc]],
                             rows[buf].at[pl.ds(s_loc * _BBLK, _BBLK)],
                             gsem[buf])

    def wait_gathers(buf):
        for s_loc in range(_CH_S):
            pltpu.make_async_copy(tok_hbm.at[idx[buf].at[s_loc]],
                                  rows[buf].at[pl.ds(s_loc * _BBLK, _BBLK)],
                                  gsem[buf]).wait()

    def fire_scatter(c, buf):
        pltpu.async_copy(trans[buf].at[:, :, :, :, pl.ds(0, _BBLK)],
                         out_hbm.at[pl.ds(c * _CH_S, _CH_S), :,
                                    pl.ds(wid, 1)],
                         ssem[buf])

    def wait_scatter(buf):
        pltpu.make_async_copy(trans[buf].at[:, :, :, :, pl.ds(0, _BBLK)],
                              out_hbm.at[pl.ds(0, _CH_S), :, pl.ds(0, 1)],
                              ssem[buf]).wait()

    zeros16 = jnp.zeros((_LANES,), jnp.int32)
    lane = lax.iota(jnp.int32, _LANES)
    i_s = [zeros16 + s_loc for s_loc in range(_CH_S)]
    i_et = []
    i_e8 = []
    for eg in range(_EG):
        e_vec = lane + eg * _LANES
        i_et.append(lax.shift_right_logical(e_vec, 3))
        i_e8.append(lax.bitwise_and(e_vec, 7))

    def compute(c, buf):
        r = rows[buf]
        t = trans[buf]
        for s_loc in range(_CH_S):
            s = c * _CH_S + s_loc
            for eg in range(_EG):
                sl = pl.ds(eg * _LANES, _LANES)

                @plsc.parallel_loop(0, _BBLK, 1, unroll=8)
                def body(b, s=s, s_loc=s_loc, eg=eg, sl=sl):
                    pv = pos_v[s, sl]
                    v = r[s_loc * _BBLK + b, sl] + pv
                    plsc.store_scatter(
                        t, [i_s[s_loc], i_et[eg], zeros16, i_e8[eg],
                            zeros16 + b], v)

    def step(c, buf, fire_next_idx=True, wait_sc=True):
        obuf = 1 - buf
        wait_gathers(obuf)
        wait_idx(buf)
        fire_gathers(buf)
        if fire_next_idx:
            fire_idx(c + 1, obuf)
        if wait_sc:
            wait_scatter(obuf)
        compute(c - 1, obuf)
        fire_scatter(c - 1, obuf)

    fire_idx(0, 0)
    wait_idx(0)
    fire_idx(1, 1)
    fire_gathers(0)
    step(1, 1, wait_sc=False)
    step(2, 0, wait_sc=False)

    def super_body(i, carry):
        step(2 * i + 1, 1)
        step(2 * i + 2, 0)
        return carry

    lax.fori_loop(1, _CHUNKS // 2 - 1, super_body, 0)

    step(_CHUNKS - 1, 1, fire_next_idx=False)
    wait_gathers(1)
    wait_scatter(1)
    compute(_CHUNKS - 1, 1)
    fire_scatter(_CHUNKS - 1, 1)
    wait_scatter(0)
    wait_scatter(1)


def kernel(inputs, token_table, pos_table):
    idx = jnp.minimum(jnp.maximum(inputs, 0), VOCAB - 1).T.reshape(-1)
    out5 = _emb_kernel(idx, token_table, pos_table)
    return out5.transpose(2, 4, 0, 1, 3).reshape(BATCH, SEQ, EMBED)
